# Initial kernel scaffold; baseline (speedup 1.0000x reference)
import jax
import jax.numpy as jnp
from jax.experimental import pallas as pl


def kernel(x, Wq, W_pk, pk_keys, keys_table, values_table, Wo):
    raise NotImplementedError("write your pallas kernel here")


# speedup vs baseline: 2.9659x; 2.9659x over previous
import functools

import jax
import jax.numpy as jnp
from jax import lax
from jax.experimental import pallas as pl
from jax.experimental.pallas import tpu as pltpu
from jax.experimental.pallas import tpu_sc as plsc

DIM = 768
HEADS = 8
NKV = 10000
TOPK = 16
PK_NUM_KEYS = 100
PK_HEADS = 8
DIM_KEY = 384
PK_TOPK = 16
S = 2048

F32 = jnp.float32
I32 = jnp.int32

_HIGH = lax.Precision.HIGHEST
BF16 = jnp.bfloat16


def _dot_bf16(a, b, dims):
    return lax.dot_general(a.astype(BF16), b.astype(BF16), (dims, ((), ())),
                           preferred_element_type=F32)


_PROJ_BLK = 1024


def _proj_body(x_ref, w_ref, o_ref):
    o_ref[...] = _dot_bf16(x_ref[...], w_ref[...], ((1,), (0,)))


_PROJ_RBLK = 512


def _projection(x2d, w):
    n = w.shape[1]
    return pl.pallas_call(
        _proj_body,
        grid=(S // _PROJ_RBLK, n // _PROJ_BLK),
        in_specs=[
            pl.BlockSpec((_PROJ_RBLK, DIM), lambda i, j: (i, 0)),
            pl.BlockSpec((DIM, _PROJ_BLK), lambda i, j: (0, j)),
        ],
        out_specs=pl.BlockSpec((_PROJ_RBLK, _PROJ_BLK), lambda i, j: (i, j)),
        out_shape=jax.ShapeDtypeStruct((S, n), F32),
    )(x2d, w)


_PK_SBLK = 128
_NEG = float("-inf")


def _topk_cols(scores, k, ncols):
    bi = lax.broadcasted_iota(I32, scores.shape, 1)
    vals, idxs = [], []
    for _ in range(k):
        m = jnp.max(scores, axis=-1, keepdims=True)
        sel = jnp.min(jnp.where(scores == m, bi, ncols), axis=-1, keepdims=True)
        vals.append(m)
        idxs.append(sel)
        scores = jnp.where(bi == sel, _NEG, scores)
    return jnp.concatenate(vals, axis=1), jnp.concatenate(idxs, axis=1)


def _pk_body(xpk_ref, pkk_ref, w_ref, i_ref):
    s_list, i_list = [], []
    for ph in range(2 * PK_HEADS):
        p, h = ph // PK_HEADS, ph % PK_HEADS
        q = xpk_ref[:, ph * DIM_KEY:(ph + 1) * DIM_KEY]
        keys = pkk_ref[p, :, h, :]
        sc = _dot_bf16(q, keys, ((1,), (1,)))
        sv, si = _topk_cols(sc, PK_TOPK, PK_NUM_KEYS)
        s_list.append(sv)
        i_list.append(si)

    sb = xpk_ref.shape[0]
    shp = (sb, PK_TOPK * PK_TOPK)
    pos = lax.broadcasted_iota(I32, shp, 1)
    er = lax.broadcasted_iota(I32, (PK_TOPK, PK_TOPK * PK_TOPK), 0)
    ec = lax.broadcasted_iota(I32, (PK_TOPK, PK_TOPK * PK_TOPK), 1)
    E = (er == ec // PK_TOPK).astype(F32)
    T = (er == ec % PK_TOPK).astype(F32)

    def _expand(a, m):
        return lax.dot_general(a, m, (((1,), (0,)), ((), ())),
                               preferred_element_type=F32, precision=_HIGH)

    for h in range(HEADS):
        s0, i0 = s_list[h], i_list[h]
        s1, i1 = s_list[PK_HEADS + h], i_list[PK_HEADS + h]
        c = _expand(s0, E) + _expand(s1, T)
        ci = (_expand(i0.astype(F32), E)
              + _expand(i1.astype(F32), T) * PK_NUM_KEYS)
        svals, sidxs = [], []
        for _ in range(TOPK):
            m = jnp.max(c, axis=-1, keepdims=True)
            sel = jnp.min(jnp.where(c == m, pos, 256), axis=-1, keepdims=True)
            hit = pos == sel
            svals.append(m)
            sidxs.append(jnp.sum(jnp.where(hit, ci, 0.0), axis=-1,
                                 keepdims=True))
            c = jnp.where(hit, _NEG, c)
        sv = jnp.concatenate(svals, axis=1)
        si = jnp.concatenate(sidxs, axis=1).astype(I32)
        mx = jnp.max(sv, axis=-1, keepdims=True)
        e = jnp.exp(sv - mx)
        w = e / jnp.sum(e, axis=-1, keepdims=True)
        w_ref[:, h, :] = _expand(w, E)
        i_ref[:, h, :] = si + h * NKV


def _pk_routing(xpk, pk_keys):
    return pl.pallas_call(
        _pk_body,
        grid=(S // _PK_SBLK,),
        in_specs=[
            pl.BlockSpec((_PK_SBLK, 2 * PK_HEADS * DIM_KEY), lambda i: (i, 0)),
            pl.BlockSpec((2, PK_NUM_KEYS, PK_HEADS, DIM_KEY),
                         lambda i: (0, 0, 0, 0)),
        ],
        out_specs=[
            pl.BlockSpec((_PK_SBLK, HEADS, TOPK * TOPK), lambda i: (i, 0, 0)),
            pl.BlockSpec((_PK_SBLK, HEADS, TOPK), lambda i: (i, 0, 0)),
        ],
        out_shape=[
            jax.ShapeDtypeStruct((S, HEADS, TOPK * TOPK), F32),
            jax.ShapeDtypeStruct((S, HEADS, TOPK), I32),
        ],
    )(xpk, pk_keys)


_NC, _NS, _L = 2, 16, 16
_NW = _NC * _NS
_ROWS = S * HEADS
_PER_W = _ROWS // _NW
_NB = 2
_NCH = _PER_W // _NB


def _sc_body(kt_ref, vt_ref, idx_ref, w_ref, ko_ref, vo_ref,
             idxv, wbuf, gbuf, obuf, gsem):
    wid = lax.axis_index("s") * _NC + lax.axis_index("c")
    base = wid * _PER_W
    pltpu.sync_copy(idx_ref.at[pl.ds(base * TOPK, _PER_W * TOPK)], idxv)

    for table_ref, out_ref in ((kt_ref, ko_ref), (vt_ref, vo_ref)):
        def chunk_body(c, _, table_ref=table_ref, out_ref=out_ref):
            dma = pltpu.make_async_copy(
                table_ref.at[idxv.at[pl.ds(c * (_NB * TOPK), _NB * TOPK)]],
                gbuf, gsem)
            dma.start()
            pltpu.sync_copy(
                w_ref.at[pl.ds((base + c * _NB) * TOPK, _NB * TOPK)], wbuf)
            dma.wait()

            def bag_body(b, _):
                ws = [wbuf[b * TOPK + j, :] for j in range(TOPK)]

                def d_body(d, _):
                    col = d * _L
                    acc0 = ws[0] * gbuf[b * TOPK + 0, pl.ds(col, _L)]
                    acc1 = ws[1] * gbuf[b * TOPK + 1, pl.ds(col, _L)]
                    acc2 = ws[2] * gbuf[b * TOPK + 2, pl.ds(col, _L)]
                    acc3 = ws[3] * gbuf[b * TOPK + 3, pl.ds(col, _L)]
                    for j in range(4, TOPK, 4):
                        acc0 += ws[j] * gbuf[b * TOPK + j, pl.ds(col, _L)]
                        acc1 += ws[j + 1] * gbuf[b * TOPK + j + 1, pl.ds(col, _L)]
                        acc2 += ws[j + 2] * gbuf[b * TOPK + j + 2, pl.ds(col, _L)]
                        acc3 += ws[j + 3] * gbuf[b * TOPK + j + 3, pl.ds(col, _L)]
                    obuf[b, pl.ds(col, _L)] = (acc0 + acc1) + (acc2 + acc3)
                    return ()

                lax.fori_loop(0, DIM // _L, d_body, ())
                return ()

            lax.fori_loop(0, _NB, bag_body, ())
            pltpu.sync_copy(obuf, out_ref.at[pl.ds(base + c * _NB, _NB)])
            return ()

        lax.fori_loop(0, _NCH, chunk_body, ())


def _sc_gather_combine(keys_table, values_table, idx_flat, w_exp):
    mesh = plsc.VectorSubcoreMesh(core_axis_name="c", subcore_axis_name="s")
    fn = functools.partial(
        pl.kernel,
        out_type=[
            jax.ShapeDtypeStruct((_ROWS, DIM), F32),
            jax.ShapeDtypeStruct((_ROWS, DIM), F32),
        ],
        mesh=mesh,
        scratch_types=[
            pltpu.VMEM((_PER_W * TOPK,), I32),
            pltpu.VMEM((_NB * TOPK, _L), F32),
            pltpu.VMEM((_NB * TOPK, DIM), F32),
            pltpu.VMEM((_NB, DIM), F32),
            pltpu.SemaphoreType.DMA,
        ],
    )(_sc_body)
    return fn(keys_table, values_table, idx_flat, w_exp)


_ATT_SBLK = 128


def _att_body(q_ref, k_ref, v_ref, o_ref):
    i = pl.program_id(1)
    q = q_ref[...] * (DIM ** -0.5)
    k = k_ref[0, :, :]
    v = v_ref[0, :, :]
    sim = _dot_bf16(q, k, ((1,), (1,)))
    rows = i * _ATT_SBLK + lax.broadcasted_iota(I32, sim.shape, 0)
    cols = lax.broadcasted_iota(I32, sim.shape, 1)
    sim = jnp.where(cols > rows, jnp.finfo(F32).min, sim)
    m = jnp.max(sim, axis=-1, keepdims=True)
    p = jnp.exp(sim - m)
    attn = p / jnp.sum(p, axis=-1, keepdims=True)
    o_ref[0, :, :] = _dot_bf16(attn, v, ((1,), (0,)))


def _attention(q2d, k3, v3):
    return pl.pallas_call(
        _att_body,
        grid=(HEADS, S // _ATT_SBLK),
        in_specs=[
            pl.BlockSpec((_ATT_SBLK, DIM), lambda h, i: (i, h)),
            pl.BlockSpec((1, S, DIM), lambda h, i: (h, 0, 0)),
            pl.BlockSpec((1, S, DIM), lambda h, i: (h, 0, 0)),
        ],
        out_specs=pl.BlockSpec((1, _ATT_SBLK, DIM), lambda h, i: (h, i, 0)),
        out_shape=jax.ShapeDtypeStruct((HEADS, S, DIM), F32),
    )(q2d, k3, v3)


def _wo_body(ao_ref, wo_ref, o_ref):
    h = pl.program_id(1)

    @pl.when(h == 0)
    def _():
        o_ref[...] = jnp.zeros_like(o_ref)

    o_ref[...] += _dot_bf16(ao_ref[0], wo_ref[0], ((1,), (0,)))


def _out_proj(ao, wo3):
    return pl.pallas_call(
        _wo_body,
        grid=(S // _ATT_SBLK, HEADS),
        in_specs=[
            pl.BlockSpec((1, _ATT_SBLK, DIM), lambda i, h: (h, i, 0)),
            pl.BlockSpec((1, DIM, DIM), lambda i, h: (h, 0, 0)),
        ],
        out_specs=pl.BlockSpec((_ATT_SBLK, DIM), lambda i, h: (i, 0)),
        out_shape=jax.ShapeDtypeStruct((S, DIM), F32),
    )(ao, wo3)


def kernel(x, Wq, W_pk, pk_keys, keys_table, values_table, Wo):
    b, s, _ = x.shape
    x2d = x.reshape(S, DIM)

    w_all = jnp.concatenate([Wq, W_pk], axis=1)
    xw = _projection(x2d, w_all)
    xq = xw[:, :DIM * HEADS]
    xpk = xw[:, DIM * HEADS:]

    weights, indices = _pk_routing(xpk, pk_keys)

    idx_flat = indices.transpose(1, 0, 2).reshape(_ROWS * TOPK)
    w_exp = weights.transpose(1, 0, 2).reshape(_ROWS * TOPK, TOPK)
    kc, vc = _sc_gather_combine(keys_table, values_table, idx_flat, w_exp)

    k3 = kc.reshape(HEADS, S, DIM)
    v3 = vc.reshape(HEADS, S, DIM)

    ao = _attention(xq, k3, v3)

    wo3 = Wo.reshape(HEADS, DIM, DIM)
    out = _out_proj(ao, wo3)
    return out.reshape(b, s, DIM)


# speedup vs baseline: 3.7473x; 1.2635x over previous
import functools

import jax
import jax.numpy as jnp
from jax import lax
from jax.experimental import pallas as pl
from jax.experimental.pallas import tpu as pltpu
from jax.experimental.pallas import tpu_sc as plsc

DIM = 768
HEADS = 8
NKV = 10000
TOPK = 16
PK_NUM_KEYS = 100
PK_HEADS = 8
DIM_KEY = 384
PK_TOPK = 16
S = 2048

F32 = jnp.float32
I32 = jnp.int32

_HIGH = lax.Precision.HIGHEST
BF16 = jnp.bfloat16


def _dot_bf16(a, b, dims):
    return lax.dot_general(a.astype(BF16), b.astype(BF16), (dims, ((), ())),
                           preferred_element_type=F32)


_PROJ_BLK = 1024


def _proj_body(x_ref, w_ref, o_ref):
    o_ref[...] = _dot_bf16(x_ref[...], w_ref[...], ((1,), (0,)))


_PROJ_RBLK = 512


def _projection(x2d, w):
    n = w.shape[1]
    return pl.pallas_call(
        _proj_body,
        grid=(S // _PROJ_RBLK, n // _PROJ_BLK),
        in_specs=[
            pl.BlockSpec((_PROJ_RBLK, DIM), lambda i, j: (i, 0)),
            pl.BlockSpec((DIM, _PROJ_BLK), lambda i, j: (0, j)),
        ],
        out_specs=pl.BlockSpec((_PROJ_RBLK, _PROJ_BLK), lambda i, j: (i, j)),
        out_shape=jax.ShapeDtypeStruct((S, n), F32),
    )(x2d, w)


_PK_SBLK = 128
_NEG = float("-inf")


def _topk_cols(scores, k, ncols):
    bi = lax.broadcasted_iota(I32, scores.shape, 1)
    vals, idxs = [], []
    for _ in range(k):
        m = jnp.max(scores, axis=-1, keepdims=True)
        sel = jnp.min(jnp.where(scores == m, bi, ncols), axis=-1, keepdims=True)
        vals.append(m)
        idxs.append(sel)
        scores = jnp.where(bi == sel, _NEG, scores)
    return jnp.concatenate(vals, axis=1), jnp.concatenate(idxs, axis=1)


def _pk_body(xpk_ref, pkk_ref, w_ref, i_ref):
    s_list, i_list = [], []
    for ph in range(2 * PK_HEADS):
        p, h = ph // PK_HEADS, ph % PK_HEADS
        q = xpk_ref[:, ph * DIM_KEY:(ph + 1) * DIM_KEY]
        keys = pkk_ref[p, :, h, :]
        sc = _dot_bf16(q, keys, ((1,), (1,)))
        sv, si = _topk_cols(sc, PK_TOPK, PK_NUM_KEYS)
        s_list.append(sv)
        i_list.append(si)

    sb = xpk_ref.shape[0]
    shp = (sb, PK_TOPK * PK_TOPK)
    pos = lax.broadcasted_iota(I32, shp, 1)
    er = lax.broadcasted_iota(I32, (PK_TOPK, PK_TOPK * PK_TOPK), 0)
    ec = lax.broadcasted_iota(I32, (PK_TOPK, PK_TOPK * PK_TOPK), 1)
    E = (er == ec // PK_TOPK).astype(F32)
    T = (er == ec % PK_TOPK).astype(F32)

    def _expand(a, m):
        return lax.dot_general(a, m, (((1,), (0,)), ((), ())),
                               preferred_element_type=F32, precision=_HIGH)

    for h in range(HEADS):
        s0, i0 = s_list[h], i_list[h]
        s1, i1 = s_list[PK_HEADS + h], i_list[PK_HEADS + h]
        c = _expand(s0, E) + _expand(s1, T)
        ci = (_expand(i0.astype(F32), E)
              + _expand(i1.astype(F32), T) * PK_NUM_KEYS)
        svals, sidxs = [], []
        for _ in range(TOPK):
            m = jnp.max(c, axis=-1, keepdims=True)
            sel = jnp.min(jnp.where(c == m, pos, 256), axis=-1, keepdims=True)
            hit = pos == sel
            svals.append(m)
            sidxs.append(jnp.sum(jnp.where(hit, ci, 0.0), axis=-1,
                                 keepdims=True))
            c = jnp.where(hit, _NEG, c)
        sv = jnp.concatenate(svals, axis=1)
        si = jnp.concatenate(sidxs, axis=1).astype(I32)
        mx = jnp.max(sv, axis=-1, keepdims=True)
        e = jnp.exp(sv - mx)
        w = e / jnp.sum(e, axis=-1, keepdims=True)
        w_ref[:, h, :] = _expand(w, E)
        i_ref[:, h, :] = si + h * NKV


def _pk_routing(xpk, pk_keys):
    return pl.pallas_call(
        _pk_body,
        grid=(S // _PK_SBLK,),
        in_specs=[
            pl.BlockSpec((_PK_SBLK, 2 * PK_HEADS * DIM_KEY), lambda i: (i, 0)),
            pl.BlockSpec((2, PK_NUM_KEYS, PK_HEADS, DIM_KEY),
                         lambda i: (0, 0, 0, 0)),
        ],
        out_specs=[
            pl.BlockSpec((_PK_SBLK, HEADS, TOPK * TOPK), lambda i: (i, 0, 0)),
            pl.BlockSpec((_PK_SBLK, HEADS, TOPK), lambda i: (i, 0, 0)),
        ],
        out_shape=[
            jax.ShapeDtypeStruct((S, HEADS, TOPK * TOPK), F32),
            jax.ShapeDtypeStruct((S, HEADS, TOPK), I32),
        ],
    )(xpk, pk_keys)


_NC, _NS, _L = 2, 16, 16
_NW = _NC * _NS
_ROWS = S * HEADS
_PER_W = _ROWS // _NW
_NB = 2
_NCH = _PER_W // _NB


_CHW = _NB * TOPK


def _sc_body(kt_ref, vt_ref, idx_ref, w_ref, ko_ref, vo_ref,
             idxv, wbuf, gbuf, obuf, gsem0, gsem1, wsem0, wsem1):
    wid = lax.axis_index("s") * _NC + lax.axis_index("c")
    base = wid * _PER_W
    pltpu.sync_copy(idx_ref.at[pl.ds(base * TOPK, _PER_W * TOPK)], idxv)

    gsems = (gsem0, gsem1)
    wsems = (wsem0, wsem1)

    for table_ref, out_ref in ((kt_ref, ko_ref), (vt_ref, vo_ref)):
        def fetch(c, slot, table_ref=table_ref):
            pltpu.make_async_copy(
                table_ref.at[idxv.at[pl.ds(c * _CHW, _CHW)]],
                gbuf.at[slot], gsems[slot]).start()
            pltpu.make_async_copy(
                w_ref.at[pl.ds(base * TOPK + c * _CHW, _CHW)],
                wbuf.at[slot], wsems[slot]).start()

        def wait_fetch(c, slot, table_ref=table_ref):
            pltpu.make_async_copy(
                table_ref.at[idxv.at[pl.ds(c * _CHW, _CHW)]],
                gbuf.at[slot], gsems[slot]).wait()
            pltpu.make_async_copy(
                w_ref.at[pl.ds(base * TOPK + c * _CHW, _CHW)],
                wbuf.at[slot], wsems[slot]).wait()

        def combine(c, slot, out_ref=out_ref):
            for b in range(_NB):
                ws = [wbuf[slot, b * TOPK + j, :] for j in range(TOPK)]

                def d_body(d, _, b=b, ws=ws):
                    col = d * _L
                    acc0 = ws[0] * gbuf[slot, b * TOPK + 0, pl.ds(col, _L)]
                    acc1 = ws[1] * gbuf[slot, b * TOPK + 1, pl.ds(col, _L)]
                    acc2 = ws[2] * gbuf[slot, b * TOPK + 2, pl.ds(col, _L)]
                    acc3 = ws[3] * gbuf[slot, b * TOPK + 3, pl.ds(col, _L)]
                    for j in range(4, TOPK, 4):
                        acc0 += ws[j] * gbuf[slot, b * TOPK + j, pl.ds(col, _L)]
                        acc1 += ws[j + 1] * gbuf[slot, b * TOPK + j + 1,
                                                 pl.ds(col, _L)]
                        acc2 += ws[j + 2] * gbuf[slot, b * TOPK + j + 2,
                                                 pl.ds(col, _L)]
                        acc3 += ws[j + 3] * gbuf[slot, b * TOPK + j + 3,
                                                 pl.ds(col, _L)]
                    obuf[b, pl.ds(col, _L)] = (acc0 + acc1) + (acc2 + acc3)
                    return ()

                lax.fori_loop(0, DIM // _L, d_body, ())
            pltpu.sync_copy(obuf, out_ref.at[pl.ds(base + c * _NB, _NB)])

        fetch(0, 0)

        def pair_body(cc, _):
            c0 = 2 * cc
            fetch(c0 + 1, 1)
            wait_fetch(c0, 0)
            combine(c0, 0)
            fetch(c0 + 2, 0)
            wait_fetch(c0 + 1, 1)
            combine(c0 + 1, 1)
            return ()

        lax.fori_loop(0, _NCH // 2 - 1, pair_body, ())
        fetch(_NCH - 1, 1)
        wait_fetch(_NCH - 2, 0)
        combine(_NCH - 2, 0)
        wait_fetch(_NCH - 1, 1)
        combine(_NCH - 1, 1)


def _sc_gather_combine(keys_table, values_table, idx_flat, w_exp):
    mesh = plsc.VectorSubcoreMesh(core_axis_name="c", subcore_axis_name="s")
    fn = functools.partial(
        pl.kernel,
        out_type=[
            jax.ShapeDtypeStruct((_ROWS, DIM), F32),
            jax.ShapeDtypeStruct((_ROWS, DIM), F32),
        ],
        mesh=mesh,
        scratch_types=[
            pltpu.VMEM((_PER_W * TOPK,), I32),
            pltpu.VMEM((2, _NB * TOPK, _L), F32),
            pltpu.VMEM((2, _NB * TOPK, DIM), F32),
            pltpu.VMEM((_NB, DIM), F32),
            pltpu.SemaphoreType.DMA,
            pltpu.SemaphoreType.DMA,
            pltpu.SemaphoreType.DMA,
            pltpu.SemaphoreType.DMA,
        ],
    )(_sc_body)
    return fn(keys_table, values_table, idx_flat, w_exp)


_ATT_SBLK = 128


def _att_body(q_ref, k_ref, v_ref, o_ref):
    i = pl.program_id(1)
    q = q_ref[...] * (DIM ** -0.5)
    k = k_ref[0, :, :]
    v = v_ref[0, :, :]
    sim = _dot_bf16(q, k, ((1,), (1,)))
    rows = i * _ATT_SBLK + lax.broadcasted_iota(I32, sim.shape, 0)
    cols = lax.broadcasted_iota(I32, sim.shape, 1)
    sim = jnp.where(cols > rows, jnp.finfo(F32).min, sim)
    m = jnp.max(sim, axis=-1, keepdims=True)
    p = jnp.exp(sim - m)
    attn = p / jnp.sum(p, axis=-1, keepdims=True)
    o_ref[0, :, :] = _dot_bf16(attn, v, ((1,), (0,)))


def _attention(q2d, k3, v3):
    return pl.pallas_call(
        _att_body,
        grid=(HEADS, S // _ATT_SBLK),
        in_specs=[
            pl.BlockSpec((_ATT_SBLK, DIM), lambda h, i: (i, h)),
            pl.BlockSpec((1, S, DIM), lambda h, i: (h, 0, 0)),
            pl.BlockSpec((1, S, DIM), lambda h, i: (h, 0, 0)),
        ],
        out_specs=pl.BlockSpec((1, _ATT_SBLK, DIM), lambda h, i: (h, i, 0)),
        out_shape=jax.ShapeDtypeStruct((HEADS, S, DIM), F32),
    )(q2d, k3, v3)


def _wo_body(ao_ref, wo_ref, o_ref):
    h = pl.program_id(1)

    @pl.when(h == 0)
    def _():
        o_ref[...] = jnp.zeros_like(o_ref)

    o_ref[...] += _dot_bf16(ao_ref[0], wo_ref[0], ((1,), (0,)))


def _out_proj(ao, wo3):
    return pl.pallas_call(
        _wo_body,
        grid=(S // _ATT_SBLK, HEADS),
        in_specs=[
            pl.BlockSpec((1, _ATT_SBLK, DIM), lambda i, h: (h, i, 0)),
            pl.BlockSpec((1, DIM, DIM), lambda i, h: (h, 0, 0)),
        ],
        out_specs=pl.BlockSpec((_ATT_SBLK, DIM), lambda i, h: (i, 0)),
        out_shape=jax.ShapeDtypeStruct((S, DIM), F32),
    )(ao, wo3)


def kernel(x, Wq, W_pk, pk_keys, keys_table, values_table, Wo):
    b, s, _ = x.shape
    x2d = x.reshape(S, DIM)

    w_all = jnp.concatenate([Wq, W_pk], axis=1)
    xw = _projection(x2d, w_all)
    xq = xw[:, :DIM * HEADS]
    xpk = xw[:, DIM * HEADS:]

    weights, indices = _pk_routing(xpk, pk_keys)

    idx_flat = indices.transpose(1, 0, 2).reshape(_ROWS * TOPK)
    w_exp = weights.transpose(1, 0, 2).reshape(_ROWS * TOPK, TOPK)
    kc, vc = _sc_gather_combine(keys_table, values_table, idx_flat, w_exp)

    k3 = kc.reshape(HEADS, S, DIM)
    v3 = vc.reshape(HEADS, S, DIM)

    ao = _attention(xq, k3, v3)

    wo3 = Wo.reshape(HEADS, DIM, DIM)
    out = _out_proj(ao, wo3)
    return out.reshape(b, s, DIM)


# speedup vs baseline: 3.7857x; 1.0102x over previous
import functools

import jax
import jax.numpy as jnp
from jax import lax
from jax.experimental import pallas as pl
from jax.experimental.pallas import tpu as pltpu
from jax.experimental.pallas import tpu_sc as plsc

DIM = 768
HEADS = 8
NKV = 10000
TOPK = 16
PK_NUM_KEYS = 100
PK_HEADS = 8
DIM_KEY = 384
PK_TOPK = 16
S = 2048

F32 = jnp.float32
I32 = jnp.int32

_HIGH = lax.Precision.HIGHEST
BF16 = jnp.bfloat16


def _dot_bf16(a, b, dims):
    return lax.dot_general(a.astype(BF16), b.astype(BF16), (dims, ((), ())),
                           preferred_element_type=F32)


_PROJ_BLK = 1024


def _proj_body(x_ref, w_ref, o_ref):
    o_ref[...] = _dot_bf16(x_ref[...], w_ref[...], ((1,), (0,)))


_PROJ_RBLK = 512


def _projection(x2d, w):
    n = w.shape[1]
    return pl.pallas_call(
        _proj_body,
        grid=(S // _PROJ_RBLK, n // _PROJ_BLK),
        in_specs=[
            pl.BlockSpec((_PROJ_RBLK, DIM), lambda i, j: (i, 0)),
            pl.BlockSpec((DIM, _PROJ_BLK), lambda i, j: (0, j)),
        ],
        out_specs=pl.BlockSpec((_PROJ_RBLK, _PROJ_BLK), lambda i, j: (i, j)),
        out_shape=jax.ShapeDtypeStruct((S, n), F32),
    )(x2d, w)


_PK_SBLK = 128
_NEG = float("-inf")


def _topk_cols(scores, k, ncols):
    bi = lax.broadcasted_iota(I32, scores.shape, 1)
    vals, idxs = [], []
    for _ in range(k):
        m = jnp.max(scores, axis=-1, keepdims=True)
        sel = jnp.min(jnp.where(scores == m, bi, ncols), axis=-1, keepdims=True)
        vals.append(m)
        idxs.append(sel)
        scores = jnp.where(bi == sel, _NEG, scores)
    return jnp.concatenate(vals, axis=1), jnp.concatenate(idxs, axis=1)


def _pk_body(xpk_ref, pkk_ref, w_ref, i_ref):
    s_list, i_list = [], []
    for ph in range(2 * PK_HEADS):
        p, h = ph // PK_HEADS, ph % PK_HEADS
        q = xpk_ref[:, ph * DIM_KEY:(ph + 1) * DIM_KEY]
        keys = pkk_ref[p, :, h, :]
        sc = _dot_bf16(q, keys, ((1,), (1,)))
        sv, si = _topk_cols(sc, PK_TOPK, PK_NUM_KEYS)
        s_list.append(sv)
        i_list.append(si)

    sb = xpk_ref.shape[0]
    shp = (sb, PK_TOPK * PK_TOPK)
    pos = lax.broadcasted_iota(I32, shp, 1)
    er = lax.broadcasted_iota(I32, (PK_TOPK, PK_TOPK * PK_TOPK), 0)
    ec = lax.broadcasted_iota(I32, (PK_TOPK, PK_TOPK * PK_TOPK), 1)
    E = (er == ec // PK_TOPK).astype(F32)
    T = (er == ec % PK_TOPK).astype(F32)

    def _expand(a, m):
        return lax.dot_general(a, m, (((1,), (0,)), ((), ())),
                               preferred_element_type=F32, precision=_HIGH)

    for h in range(HEADS):
        s0, i0 = s_list[h], i_list[h]
        s1, i1 = s_list[PK_HEADS + h], i_list[PK_HEADS + h]
        c = _expand(s0, E) + _expand(s1, T)
        ci = (_expand(i0.astype(F32), E)
              + _expand(i1.astype(F32), T) * PK_NUM_KEYS)
        svals, sidxs = [], []
        for _ in range(TOPK):
            m = jnp.max(c, axis=-1, keepdims=True)
            sel = jnp.min(jnp.where(c == m, pos, 256), axis=-1, keepdims=True)
            hit = pos == sel
            svals.append(m)
            sidxs.append(jnp.sum(jnp.where(hit, ci, 0.0), axis=-1,
                                 keepdims=True))
            c = jnp.where(hit, _NEG, c)
        sv = jnp.concatenate(svals, axis=1)
        si = jnp.concatenate(sidxs, axis=1).astype(I32)
        mx = jnp.max(sv, axis=-1, keepdims=True)
        e = jnp.exp(sv - mx)
        w = e / jnp.sum(e, axis=-1, keepdims=True)
        w_ref[:, h, :] = _expand(w, E)
        i_ref[:, h, :] = si + h * NKV


def _pk_routing(xpk, pk_keys):
    return pl.pallas_call(
        _pk_body,
        grid=(S // _PK_SBLK,),
        in_specs=[
            pl.BlockSpec((_PK_SBLK, 2 * PK_HEADS * DIM_KEY), lambda i: (i, 0)),
            pl.BlockSpec((2, PK_NUM_KEYS, PK_HEADS, DIM_KEY),
                         lambda i: (0, 0, 0, 0)),
        ],
        out_specs=[
            pl.BlockSpec((_PK_SBLK, HEADS, TOPK * TOPK), lambda i: (i, 0, 0)),
            pl.BlockSpec((_PK_SBLK, HEADS, TOPK), lambda i: (i, 0, 0)),
        ],
        out_shape=[
            jax.ShapeDtypeStruct((S, HEADS, TOPK * TOPK), F32),
            jax.ShapeDtypeStruct((S, HEADS, TOPK), I32),
        ],
    )(xpk, pk_keys)


_NC, _NS, _L = 2, 16, 16
_NW = _NC * _NS
_ROWS = S * HEADS
_PER_W = _ROWS // _NW
_NB = 4
_NCH = _PER_W // _NB


_CHW = _NB * TOPK


def _sc_body(kt_ref, vt_ref, idx_ref, w_ref, ko_ref, vo_ref,
             idxv, wbuf, gbuf, obuf, gsem0, gsem1, wsem0, wsem1):
    wid = lax.axis_index("s") * _NC + lax.axis_index("c")
    base = wid * _PER_W
    pltpu.sync_copy(idx_ref.at[pl.ds(base * TOPK, _PER_W * TOPK)], idxv)

    gsems = (gsem0, gsem1)
    wsems = (wsem0, wsem1)

    for table_ref, out_ref in ((kt_ref, ko_ref), (vt_ref, vo_ref)):
        def fetch(c, slot, table_ref=table_ref):
            pltpu.make_async_copy(
                table_ref.at[idxv.at[pl.ds(c * _CHW, _CHW)]],
                gbuf.at[slot], gsems[slot]).start()
            pltpu.make_async_copy(
                w_ref.at[pl.ds(base * TOPK + c * _CHW, _CHW)],
                wbuf.at[slot], wsems[slot]).start()

        def wait_fetch(c, slot, table_ref=table_ref):
            pltpu.make_async_copy(
                table_ref.at[idxv.at[pl.ds(c * _CHW, _CHW)]],
                gbuf.at[slot], gsems[slot]).wait()
            pltpu.make_async_copy(
                w_ref.at[pl.ds(base * TOPK + c * _CHW, _CHW)],
                wbuf.at[slot], wsems[slot]).wait()

        def combine(c, slot, out_ref=out_ref):
            for b in range(_NB):
                ws = [wbuf[slot, b * TOPK + j, :] for j in range(TOPK)]

                def d_body(d, _, b=b, ws=ws):
                    col = d * _L
                    acc0 = ws[0] * gbuf[slot, b * TOPK + 0, pl.ds(col, _L)]
                    acc1 = ws[1] * gbuf[slot, b * TOPK + 1, pl.ds(col, _L)]
                    acc2 = ws[2] * gbuf[slot, b * TOPK + 2, pl.ds(col, _L)]
                    acc3 = ws[3] * gbuf[slot, b * TOPK + 3, pl.ds(col, _L)]
                    for j in range(4, TOPK, 4):
                        acc0 += ws[j] * gbuf[slot, b * TOPK + j, pl.ds(col, _L)]
                        acc1 += ws[j + 1] * gbuf[slot, b * TOPK + j + 1,
                                                 pl.ds(col, _L)]
                        acc2 += ws[j + 2] * gbuf[slot, b * TOPK + j + 2,
                                                 pl.ds(col, _L)]
                        acc3 += ws[j + 3] * gbuf[slot, b * TOPK + j + 3,
                                                 pl.ds(col, _L)]
                    obuf[b, pl.ds(col, _L)] = (acc0 + acc1) + (acc2 + acc3)
                    return ()

                lax.fori_loop(0, DIM // _L, d_body, ())
            pltpu.sync_copy(obuf, out_ref.at[pl.ds(base + c * _NB, _NB)])

        fetch(0, 0)

        def pair_body(cc, _):
            c0 = 2 * cc
            fetch(c0 + 1, 1)
            wait_fetch(c0, 0)
            combine(c0, 0)
            fetch(c0 + 2, 0)
            wait_fetch(c0 + 1, 1)
            combine(c0 + 1, 1)
            return ()

        lax.fori_loop(0, _NCH // 2 - 1, pair_body, ())
        fetch(_NCH - 1, 1)
        wait_fetch(_NCH - 2, 0)
        combine(_NCH - 2, 0)
        wait_fetch(_NCH - 1, 1)
        combine(_NCH - 1, 1)


def _sc_gather_combine(keys_table, values_table, idx_flat, w_exp):
    mesh = plsc.VectorSubcoreMesh(core_axis_name="c", subcore_axis_name="s")
    fn = functools.partial(
        pl.kernel,
        out_type=[
            jax.ShapeDtypeStruct((_ROWS, DIM), F32),
            jax.ShapeDtypeStruct((_ROWS, DIM), F32),
        ],
        mesh=mesh,
        scratch_types=[
            pltpu.VMEM((_PER_W * TOPK,), I32),
            pltpu.VMEM((2, _NB * TOPK, _L), F32),
            pltpu.VMEM((2, _NB * TOPK, DIM), F32),
            pltpu.VMEM((_NB, DIM), F32),
            pltpu.SemaphoreType.DMA,
            pltpu.SemaphoreType.DMA,
            pltpu.SemaphoreType.DMA,
            pltpu.SemaphoreType.DMA,
        ],
    )(_sc_body)
    return fn(keys_table, values_table, idx_flat, w_exp)


_ATT_SBLK = 128


def _att_body(q_ref, k_ref, v_ref, o_ref):
    i = pl.program_id(1)
    q = q_ref[...] * (DIM ** -0.5)
    k = k_ref[0, :, :]
    v = v_ref[0, :, :]
    sim = _dot_bf16(q, k, ((1,), (1,)))
    rows = i * _ATT_SBLK + lax.broadcasted_iota(I32, sim.shape, 0)
    cols = lax.broadcasted_iota(I32, sim.shape, 1)
    sim = jnp.where(cols > rows, jnp.finfo(F32).min, sim)
    m = jnp.max(sim, axis=-1, keepdims=True)
    p = jnp.exp(sim - m)
    attn = p / jnp.sum(p, axis=-1, keepdims=True)
    o_ref[0, :, :] = _dot_bf16(attn, v, ((1,), (0,)))


def _attention(q2d, k3, v3):
    return pl.pallas_call(
        _att_body,
        grid=(HEADS, S // _ATT_SBLK),
        in_specs=[
            pl.BlockSpec((_ATT_SBLK, DIM), lambda h, i: (i, h)),
            pl.BlockSpec((1, S, DIM), lambda h, i: (h, 0, 0)),
            pl.BlockSpec((1, S, DIM), lambda h, i: (h, 0, 0)),
        ],
        out_specs=pl.BlockSpec((1, _ATT_SBLK, DIM), lambda h, i: (h, i, 0)),
        out_shape=jax.ShapeDtypeStruct((HEADS, S, DIM), F32),
    )(q2d, k3, v3)


def _wo_body(ao_ref, wo_ref, o_ref):
    h = pl.program_id(1)

    @pl.when(h == 0)
    def _():
        o_ref[...] = jnp.zeros_like(o_ref)

    o_ref[...] += _dot_bf16(ao_ref[0], wo_ref[0], ((1,), (0,)))


def _out_proj(ao, wo3):
    return pl.pallas_call(
        _wo_body,
        grid=(S // _ATT_SBLK, HEADS),
        in_specs=[
            pl.BlockSpec((1, _ATT_SBLK, DIM), lambda i, h: (h, i, 0)),
            pl.BlockSpec((1, DIM, DIM), lambda i, h: (h, 0, 0)),
        ],
        out_specs=pl.BlockSpec((_ATT_SBLK, DIM), lambda i, h: (i, 0)),
        out_shape=jax.ShapeDtypeStruct((S, DIM), F32),
    )(ao, wo3)


def kernel(x, Wq, W_pk, pk_keys, keys_table, values_table, Wo):
    b, s, _ = x.shape
    x2d = x.reshape(S, DIM)

    w_all = jnp.concatenate([Wq, W_pk], axis=1)
    xw = _projection(x2d, w_all)
    xq = xw[:, :DIM * HEADS]
    xpk = xw[:, DIM * HEADS:]

    weights, indices = _pk_routing(xpk, pk_keys)

    idx_flat = indices.transpose(1, 0, 2).reshape(_ROWS * TOPK)
    w_exp = weights.transpose(1, 0, 2).reshape(_ROWS * TOPK, TOPK)
    kc, vc = _sc_gather_combine(keys_table, values_table, idx_flat, w_exp)

    k3 = kc.reshape(HEADS, S, DIM)
    v3 = vc.reshape(HEADS, S, DIM)

    ao = _attention(xq, k3, v3)

    wo3 = Wo.reshape(HEADS, DIM, DIM)
    out = _out_proj(ao, wo3)
    return out.reshape(b, s, DIM)
